# Initial kernel scaffold; baseline (speedup 1.0000x reference)
import jax
import jax.numpy as jnp
from jax.experimental import pallas as pl


def kernel(x, target):
    raise NotImplementedError("write your pallas kernel here")


# speedup vs baseline: 2.5307x; 2.5307x over previous
import functools
import math

import jax
import jax.numpy as jnp
from jax import lax
from jax.experimental import pallas as pl
from jax.experimental.pallas import tpu as pltpu
from jax.experimental.pallas import tpu_sc as plsc

VOCAB = 32000
PAD = 0
EPS = 0.1 / (VOCAB - 2)
CONF = 0.9
ROW_CONST = CONF * math.log(CONF) + (VOCAB - 2) * EPS * math.log(EPS)

N_ROWS = 4096
BR = 256
BC = 6400
GRID_R = N_ROWS // BR
GRID_C = VOCAB // BC


def _tc_body(tgt_ref, x_ref, out_ref):
    i = pl.program_id(0)
    j = pl.program_id(1)

    @pl.when((i == 0) & (j == 0))
    def _init():
        out_ref[0, 0] = 0.0

    tgt = tgt_ref[0, 0, :]
    mask = tgt != PAD
    w = jnp.where(mask, -EPS, 0.0)
    rs = jnp.sum(x_ref[...], axis=1)
    part = jnp.sum(rs * w)

    @pl.when(j == 0)
    def _col0_terms():
        x0 = x_ref[:, 0]
        extra = jnp.sum(jnp.where(mask, ROW_CONST + EPS * x0, 0.0))
        out_ref[0, 0] += extra

    out_ref[0, 0] += part


def _tc_sweep(x, target3d):
    return pl.pallas_call(
        _tc_body,
        grid=(GRID_R, GRID_C),
        in_specs=[
            pl.BlockSpec((1, 1, BR), lambda i, j: (i, 0, 0)),
            pl.BlockSpec((BR, BC), lambda i, j: (i, j)),
        ],
        out_specs=pl.BlockSpec(memory_space=pltpu.SMEM),
        out_shape=jax.ShapeDtypeStruct((1, 1), jnp.float32),
        compiler_params=pltpu.CompilerParams(
            dimension_semantics=("arbitrary", "arbitrary"),
        ),
    )(target3d, x)


_INFO = plsc.get_sparse_core_info()
_NC, _NS, _L = _INFO.num_cores, _INFO.num_subcores, _INFO.num_lanes
_NW = _NC * _NS
_CHUNK = N_ROWS // _NW


def _sc_body(xflat_hbm, tgt_hbm, out_hbm, tgt_v, idx_v, gat_v, acc_v, sem):
    wid = lax.axis_index("s") * _NC + lax.axis_index("c")
    base = wid * _CHUNK
    pltpu.sync_copy(tgt_hbm.at[pl.ds(base, _CHUNK)], tgt_v)
    lane = lax.iota(jnp.int32, _L)
    for j in range(_CHUNK // _L):
        t = tgt_v[pl.ds(j * _L, _L)]
        rows = (base + j * _L) + lane
        idx_v[pl.ds(j * _L, _L)] = rows * VOCAB + t
    pltpu.async_copy(xflat_hbm.at[idx_v], gat_v, sem).wait()
    acc = jnp.zeros((_L,), jnp.float32)
    for j in range(_CHUNK // _L):
        t = tgt_v[pl.ds(j * _L, _L)]
        g = gat_v[pl.ds(j * _L, _L)]
        acc = acc + jnp.where(t != PAD, g, 0.0)
    acc_v[...] = acc
    pltpu.sync_copy(acc_v, out_hbm.at[pl.ds(wid * _L, _L)])


_sc_gather = functools.partial(
    pl.kernel,
    out_type=jax.ShapeDtypeStruct((_NW * _L,), jnp.float32),
    mesh=plsc.VectorSubcoreMesh(core_axis_name="c", subcore_axis_name="s"),
    scratch_types=[
        pltpu.VMEM((_CHUNK,), jnp.int32),
        pltpu.VMEM((_CHUNK,), jnp.int32),
        pltpu.VMEM((_CHUNK,), jnp.float32),
        pltpu.VMEM((_L,), jnp.float32),
        pltpu.SemaphoreType.DMA,
    ],
)(_sc_body)


@jax.jit
def kernel(x, target):
    target = target.astype(jnp.int32)
    target3d = target.reshape(GRID_R, 1, BR)
    tc_part = _tc_sweep(x, target3d)[0, 0]
    gathered = _sc_gather(x.reshape(-1), target)
    return tc_part - (CONF - EPS) * jnp.sum(gathered)
